# Initial kernel scaffold; baseline (speedup 1.0000x reference)
import functools

import jax
import jax.numpy as jnp
from jax import lax
from jax.experimental import pallas as pl
from jax.experimental.pallas import tpu as pltpu
from jax.experimental.pallas import tpu_sc as plsc

_NC = 2
_NS = 16
_NW = _NC * _NS
_K = 80


def _sc_mesh():
    return plsc.VectorSubcoreMesh(
        core_axis_name="c", subcore_axis_name="s", num_cores=_NC, num_subcores=_NS
    )


@functools.cache
def _make_hist(n, e):
    epw = e // _NW
    iters = epw // _K
    rpt = n // _NS

    @functools.partial(
        pl.kernel,
        out_type=jax.ShapeDtypeStruct((_NC, n, 16), jnp.float32),
        mesh=_sc_mesh(),
        scratch_types=[
            pltpu.VMEM((_K,), jnp.int32),
            pltpu.VMEM((_K, 16), jnp.float32),
            pltpu.VMEM_SHARED((n, 16), jnp.float32),
        ],
    )
    def hist(dst_hbm, zeros_hbm, out_hbm, idx_v, ones_v, acc_sh):
        c = lax.axis_index("c")
        s = lax.axis_index("s")
        wid = s * _NC + c
        for r in range(_K):
            ones_v[r] = jnp.ones((16,), jnp.float32)
        pltpu.sync_copy(
            zeros_hbm.at[pl.ds(s * rpt, rpt)], acc_sh.at[pl.ds(s * rpt, rpt)]
        )
        plsc.subcore_barrier()

        def body(i, carry):
            base = wid * epw + i * _K
            pltpu.sync_copy(dst_hbm.at[pl.ds(base, _K)], idx_v)
            pltpu.sync_copy(ones_v, acc_sh.at[idx_v], add=True)
            return carry

        lax.fori_loop(0, iters, body, 0)
        plsc.subcore_barrier()
        pltpu.sync_copy(
            acc_sh.at[pl.ds(s * rpt, rpt)], out_hbm.at[c, pl.ds(s * rpt, rpt)]
        )

    return hist


@functools.cache
def _make_msg(n, e, d):
    epw = e // _NW
    iters = epw // _K
    rpt = n // _NS

    @functools.partial(
        pl.kernel,
        out_type=jax.ShapeDtypeStruct((_NC, n, d), jnp.float32),
        mesh=_sc_mesh(),
        scratch_types=[
            pltpu.VMEM((_K,), jnp.int32),
            pltpu.VMEM((_K,), jnp.int32),
            pltpu.VMEM((_K, d), jnp.float32),
            pltpu.VMEM_SHARED((n, d), jnp.float32),
        ],
    )
    def msg(hs_hbm, src_hbm, dst_hbm, zeros_hbm, out_hbm, sidx_v, didx_v, rows_v, acc_sh):
        c = lax.axis_index("c")
        s = lax.axis_index("s")
        wid = s * _NC + c
        pltpu.sync_copy(
            zeros_hbm.at[pl.ds(s * rpt, rpt)], acc_sh.at[pl.ds(s * rpt, rpt)]
        )
        plsc.subcore_barrier()

        def body(i, carry):
            base = wid * epw + i * _K
            pltpu.sync_copy(src_hbm.at[pl.ds(base, _K)], sidx_v)
            pltpu.sync_copy(dst_hbm.at[pl.ds(base, _K)], didx_v)
            pltpu.sync_copy(hs_hbm.at[sidx_v], rows_v)
            pltpu.sync_copy(rows_v, acc_sh.at[didx_v], add=True)
            return carry

        lax.fori_loop(0, iters, body, 0)
        plsc.subcore_barrier()
        pltpu.sync_copy(
            acc_sh.at[pl.ds(s * rpt, rpt)], out_hbm.at[c, pl.ds(s * rpt, rpt)]
        )

    return msg


def _scale_matmul_body(x_ref, cnt_ref, w_ref, o_ref):
    cnt = cnt_ref[...]
    deg = cnt[0, :, 0:1] + cnt[1, :, 0:1] + 1.0
    dinv = lax.rsqrt(deg)
    o_ref[...] = jnp.dot(
        x_ref[...] * dinv, w_ref[...], preferred_element_type=jnp.float32
    )


def _scale_matmul(x, cnt, w):
    n, d_in = x.shape
    d_h = w.shape[1]
    bm = 1000
    grid = n // bm
    return pl.pallas_call(
        _scale_matmul_body,
        grid=(grid,),
        in_specs=[
            pl.BlockSpec((bm, d_in), lambda i: (i, 0)),
            pl.BlockSpec((_NC, bm, 16), lambda i: (0, i, 0)),
            pl.BlockSpec((d_in, d_h), lambda i: (0, 0)),
        ],
        out_specs=pl.BlockSpec((bm, d_h), lambda i: (i, 0)),
        out_shape=jax.ShapeDtypeStruct((n, d_h), jnp.float32),
    )(x, cnt, w)


def _finalize_body(s_ref, hs_ref, cnt_ref, b_ref, g_ref, be_ref, z_ref):
    cnt = cnt_ref[...]
    deg = cnt[0, :, 0:1] + cnt[1, :, 0:1] + 1.0
    dinv = lax.rsqrt(deg)
    agg = dinv * (s_ref[0] + s_ref[1] + hs_ref[...]) + b_ref[...]
    inv_n = 1.0 / agg.shape[0]
    mean = jnp.sum(agg, axis=0, keepdims=True) * inv_n
    cen = agg - mean
    var = jnp.sum(cen * cen, axis=0, keepdims=True) * inv_n
    z = cen * lax.rsqrt(var + 1e-5) * g_ref[...] + be_ref[...]
    z_ref[...] = jnp.maximum(z, 0.0)


def _finalize(s, hs, cnt, b, g, be):
    n, d = hs.shape
    return pl.pallas_call(
        _finalize_body,
        out_shape=jax.ShapeDtypeStruct((n, d), jnp.float32),
    )(s, hs, cnt, b, g, be)


def _decoder_body(zi_ref, zj_ref, o_ref):
    o_ref[...] = lax.dot_general(
        zi_ref[...],
        zj_ref[...],
        (((1,), (1,)), ((), ())),
        preferred_element_type=jnp.float32,
    )


def _decoder(z):
    n, d = z.shape
    bm = 512
    grid = pl.cdiv(n, bm)
    return pl.pallas_call(
        _decoder_body,
        grid=(grid, grid),
        in_specs=[
            pl.BlockSpec((bm, d), lambda i, j: (i, 0)),
            pl.BlockSpec((bm, d), lambda i, j: (j, 0)),
        ],
        out_specs=pl.BlockSpec((bm, bm), lambda i, j: (i, j)),
        out_shape=jax.ShapeDtypeStruct((n, n), jnp.float32),
        compiler_params=pltpu.CompilerParams(
            dimension_semantics=("parallel", "parallel")
        ),
    )(z, z)


def kernel(x, edge_index, W, b, gamma, beta):
    n, _ = x.shape
    d_h = W.shape[1]
    e = edge_index.shape[1]
    src = edge_index[0]
    dst = edge_index[1]
    zeros16 = jnp.zeros((n, 16), jnp.float32)
    zeros_d = jnp.zeros((n, d_h), jnp.float32)
    cnt = _make_hist(n, e)(dst, zeros16)
    hs = _scale_matmul(x, cnt, W)
    s = _make_msg(n, e, d_h)(hs, src, dst, zeros_d)
    z = _finalize(
        s, hs, cnt, b.reshape(1, d_h), gamma.reshape(1, d_h), beta.reshape(1, d_h)
    )
    return _decoder(z)


# speedup vs baseline: 9.7717x; 9.7717x over previous
import functools

import jax
import jax.numpy as jnp
from jax import lax
from jax.experimental import pallas as pl
from jax.experimental.pallas import tpu as pltpu
from jax.experimental.pallas import tpu_sc as plsc

_NC = 2
_NS = 16
_NW = _NC * _NS
_K = 80


def _sc_mesh():
    return plsc.VectorSubcoreMesh(
        core_axis_name="c", subcore_axis_name="s", num_cores=_NC, num_subcores=_NS
    )


@functools.cache
def _make_hist(n, e):
    epw = e // _NW
    iters = epw // _K
    rpt = (n // _NS) // 8 * 8
    tail = n - _NS * rpt

    @functools.partial(
        pl.kernel,
        out_type=jax.ShapeDtypeStruct((_NC, n, 16), jnp.float32),
        mesh=_sc_mesh(),
        scratch_types=[
            pltpu.VMEM((_K,), jnp.int32),
            pltpu.VMEM((_K, 16), jnp.float32),
            pltpu.VMEM_SHARED((n, 16), jnp.float32),
        ],
    )
    def hist(dst_hbm, zeros_hbm, out_hbm, idx_v, ones_v, acc_sh):
        c = lax.axis_index("c")
        s = lax.axis_index("s")
        wid = s * _NC + c
        for r in range(_K):
            ones_v[r] = jnp.ones((16,), jnp.float32)
        pltpu.sync_copy(
            zeros_hbm.at[pl.ds(s * rpt, rpt)], acc_sh.at[pl.ds(s * rpt, rpt)]
        )
        if tail:
            @pl.when(s == _NS - 1)
            def _():
                pltpu.sync_copy(
                    zeros_hbm.at[pl.ds(_NS * rpt, tail)],
                    acc_sh.at[pl.ds(_NS * rpt, tail)],
                )
        plsc.subcore_barrier()

        def body(i, carry):
            base = wid * epw + i * _K
            pltpu.sync_copy(dst_hbm.at[pl.ds(base, _K)], idx_v)
            pltpu.sync_copy(ones_v, acc_sh.at[idx_v], add=True)
            return carry

        lax.fori_loop(0, iters, body, 0)
        plsc.subcore_barrier()
        pltpu.sync_copy(
            acc_sh.at[pl.ds(s * rpt, rpt)], out_hbm.at[c, pl.ds(s * rpt, rpt)]
        )
        if tail:
            @pl.when(s == _NS - 1)
            def _():
                pltpu.sync_copy(
                    acc_sh.at[pl.ds(_NS * rpt, tail)],
                    out_hbm.at[c, pl.ds(_NS * rpt, tail)],
                )

    return hist


@functools.cache
def _make_msg(n, e, d):
    epw = e // _NW
    iters = epw // _K
    rpt = (n // _NS) // 8 * 8
    tail = n - _NS * rpt

    @functools.partial(
        pl.kernel,
        out_type=jax.ShapeDtypeStruct((_NC, n, d), jnp.float32),
        mesh=_sc_mesh(),
        scratch_types=[
            pltpu.VMEM((_K,), jnp.int32),
            pltpu.VMEM((_K,), jnp.int32),
            pltpu.VMEM((_K, d), jnp.float32),
            pltpu.VMEM_SHARED((n, d), jnp.float32),
        ],
    )
    def msg(hs_hbm, src_hbm, dst_hbm, zeros_hbm, out_hbm, sidx_v, didx_v, rows_v, acc_sh):
        c = lax.axis_index("c")
        s = lax.axis_index("s")
        wid = s * _NC + c
        pltpu.sync_copy(
            zeros_hbm.at[pl.ds(s * rpt, rpt)], acc_sh.at[pl.ds(s * rpt, rpt)]
        )
        if tail:
            @pl.when(s == _NS - 1)
            def _():
                pltpu.sync_copy(
                    zeros_hbm.at[pl.ds(_NS * rpt, tail)],
                    acc_sh.at[pl.ds(_NS * rpt, tail)],
                )
        plsc.subcore_barrier()

        def body(i, carry):
            base = wid * epw + i * _K
            pltpu.sync_copy(src_hbm.at[pl.ds(base, _K)], sidx_v)
            pltpu.sync_copy(dst_hbm.at[pl.ds(base, _K)], didx_v)
            pltpu.sync_copy(hs_hbm.at[sidx_v], rows_v)
            pltpu.sync_copy(rows_v, acc_sh.at[didx_v], add=True)
            return carry

        lax.fori_loop(0, iters, body, 0)
        plsc.subcore_barrier()
        pltpu.sync_copy(
            acc_sh.at[pl.ds(s * rpt, rpt)], out_hbm.at[c, pl.ds(s * rpt, rpt)]
        )
        if tail:
            @pl.when(s == _NS - 1)
            def _():
                pltpu.sync_copy(
                    acc_sh.at[pl.ds(_NS * rpt, tail)],
                    out_hbm.at[c, pl.ds(_NS * rpt, tail)],
                )

    return msg


def _scale_matmul_body(x_ref, cnt_ref, w_ref, o_ref):
    cnt = cnt_ref[...]
    deg = cnt[0, :, 0:1] + cnt[1, :, 0:1] + 1.0
    dinv = lax.rsqrt(deg)
    o_ref[...] = jnp.dot(
        x_ref[...] * dinv, w_ref[...], preferred_element_type=jnp.float32
    )


def _scale_matmul(x, cnt, w):
    n, d_in = x.shape
    d_h = w.shape[1]
    bm = 1000
    grid = n // bm
    return pl.pallas_call(
        _scale_matmul_body,
        grid=(grid,),
        in_specs=[
            pl.BlockSpec((bm, d_in), lambda i: (i, 0)),
            pl.BlockSpec((_NC, bm, 16), lambda i: (0, i, 0)),
            pl.BlockSpec((d_in, d_h), lambda i: (0, 0)),
        ],
        out_specs=pl.BlockSpec((bm, d_h), lambda i: (i, 0)),
        out_shape=jax.ShapeDtypeStruct((n, d_h), jnp.float32),
    )(x, cnt, w)


def _finalize_body(s_ref, hs_ref, cnt_ref, b_ref, g_ref, be_ref, z_ref):
    cnt = cnt_ref[...]
    deg = cnt[0, :, 0:1] + cnt[1, :, 0:1] + 1.0
    dinv = lax.rsqrt(deg)
    agg = dinv * (s_ref[0] + s_ref[1] + hs_ref[...]) + b_ref[...]
    inv_n = 1.0 / agg.shape[0]
    mean = jnp.sum(agg, axis=0, keepdims=True) * inv_n
    cen = agg - mean
    var = jnp.sum(cen * cen, axis=0, keepdims=True) * inv_n
    z = cen * lax.rsqrt(var + 1e-5) * g_ref[...] + be_ref[...]
    z_ref[...] = jnp.maximum(z, 0.0)


def _finalize(s, hs, cnt, b, g, be):
    n, d = hs.shape
    return pl.pallas_call(
        _finalize_body,
        out_shape=jax.ShapeDtypeStruct((n, d), jnp.float32),
    )(s, hs, cnt, b, g, be)


def _decoder_body(zi_ref, zj_ref, o_ref):
    o_ref[...] = lax.dot_general(
        zi_ref[...],
        zj_ref[...],
        (((1,), (1,)), ((), ())),
        preferred_element_type=jnp.float32,
    )


def _decoder(z):
    n, d = z.shape
    bm = 512
    grid = pl.cdiv(n, bm)
    return pl.pallas_call(
        _decoder_body,
        grid=(grid, grid),
        in_specs=[
            pl.BlockSpec((bm, d), lambda i, j: (i, 0)),
            pl.BlockSpec((bm, d), lambda i, j: (j, 0)),
        ],
        out_specs=pl.BlockSpec((bm, bm), lambda i, j: (i, j)),
        out_shape=jax.ShapeDtypeStruct((n, n), jnp.float32),
        compiler_params=pltpu.CompilerParams(
            dimension_semantics=("parallel", "parallel")
        ),
    )(z, z)


def kernel(x, edge_index, W, b, gamma, beta):
    n, _ = x.shape
    d_h = W.shape[1]
    e = edge_index.shape[1]
    src = edge_index[0]
    dst = edge_index[1]
    zeros16 = jnp.zeros((n, 16), jnp.float32)
    zeros_d = jnp.zeros((n, d_h), jnp.float32)
    cnt = _make_hist(n, e)(dst, zeros16)
    hs = _scale_matmul(x, cnt, W)
    s = _make_msg(n, e, d_h)(hs, src, dst, zeros_d)
    z = _finalize(
        s, hs, cnt, b.reshape(1, d_h), gamma.reshape(1, d_h), beta.reshape(1, d_h)
    )
    return _decoder(z)
